# baseline (speedup 1.0000x reference)
import functools

import jax
import jax.numpy as jnp
import numpy as np
from jax.experimental import pallas as pl

_B = 4
_N = 8192
_NPOINT = 1024
_RADIUS = 0.2
_NSAMPLE = 32
_IN_CH = 64

_M = _B * _NPOINT * _NSAMPLE
_BLK = 1024
_C = 128
_GRID = _M // _BLK


def _sqdist(src, dst):
    d = -2.0 * jnp.matmul(src, jnp.swapaxes(dst, 1, 2))
    d = d + jnp.sum(src ** 2, -1)[:, :, None]
    d = d + jnp.sum(dst ** 2, -1)[:, None, :]
    return d


def _gather_pts(points, idx):
    return jax.vmap(lambda p, i: p[i])(points, idx)


def _fps(xyz, npoint):
    b, n, _ = xyz.shape

    def body(i, state):
        distance, farthest, centroids = state
        centroids = centroids.at[:, i].set(farthest)
        cen = jnp.take_along_axis(
            xyz, jnp.broadcast_to(farthest[:, None, None], (b, 1, 3)), axis=1)
        d = jnp.sum((xyz - cen) ** 2, -1)
        distance = jnp.minimum(distance, d)
        farthest = jnp.argmax(distance, -1).astype(jnp.int32)
        return (distance, farthest, centroids)

    state = (jnp.full((b, n), 1e10, jnp.float32),
             jnp.zeros((b,), jnp.int32),
             jnp.zeros((b, npoint), jnp.int32))
    state = jax.lax.fori_loop(0, npoint, body, state)
    return state[2]


def _ball_query(radius, nsample, xyz, new_xyz):
    b, n, _ = xyz.shape
    s = new_xyz.shape[1]
    sqr = _sqdist(new_xyz, xyz)
    gidx = jnp.broadcast_to(jnp.arange(n, dtype=jnp.int32), (b, s, n))
    gidx = jnp.where(sqr > radius ** 2, n, gidx)
    gidx = jnp.sort(gidx, axis=-1)[:, :, :nsample]
    first = gidx[:, :, :1]
    first = jnp.where(first == n, 0, first)
    gidx = jnp.where(gidx == n, jnp.broadcast_to(first, gidx.shape), gidx)
    return gidx


def _mm_stats_kernel(x_ref, w_ref, y_ref, stats_ref):
    y = jnp.dot(x_ref[...], w_ref[...], preferred_element_type=jnp.float32)
    y_ref[...] = y

    @pl.when(pl.program_id(0) == 0)
    def _():
        stats_ref[...] = jnp.zeros_like(stats_ref)

    s1 = jnp.sum(y, axis=0, keepdims=True)
    s2 = jnp.sum(y * y, axis=0, keepdims=True)
    pad = jnp.zeros((6, _C), jnp.float32)
    stats_ref[...] += jnp.concatenate([s1, s2, pad], axis=0)


def _affine_mm_stats_kernel(x_ref, aff_ref, w_ref, y_ref, stats_ref):
    aff = aff_ref[...]
    x = jnp.maximum(x_ref[...] * aff[0:1, :] + aff[1:2, :], 0.0)
    y = jnp.dot(x, w_ref[...], preferred_element_type=jnp.float32)
    y_ref[...] = y

    @pl.when(pl.program_id(0) == 0)
    def _():
        stats_ref[...] = jnp.zeros_like(stats_ref)

    s1 = jnp.sum(y, axis=0, keepdims=True)
    s2 = jnp.sum(y * y, axis=0, keepdims=True)
    pad = jnp.zeros((6, _C), jnp.float32)
    stats_ref[...] += jnp.concatenate([s1, s2, pad], axis=0)


def _affine_maxpool_kernel(x_ref, aff_ref, o_ref):
    aff = aff_ref[...]
    x = jnp.maximum(x_ref[...] * aff[0:1, :] + aff[1:2, :], 0.0)
    x = x.reshape(_BLK // _NSAMPLE, _NSAMPLE, _C)
    o_ref[...] = jnp.max(x, axis=1)


def _affine_from_stats(stats, g, b):
    mean = stats[0] / _M
    var = stats[1] / _M - mean * mean
    scale = g / jnp.sqrt(var + 1e-5)
    shift = b - mean * scale
    return jnp.concatenate([scale[None, :], shift[None, :],
                            jnp.zeros((6, _C), jnp.float32)], axis=0)


def _pad_w(W):
    o, i = W.shape
    Wt = jnp.zeros((_C, _C), jnp.float32).at[:i, :o].set(W.T)
    return Wt


def _pad_vec(v):
    return jnp.zeros((_C,), jnp.float32).at[: v.shape[0]].set(v)


@functools.partial(jax.jit)
def kernel(xyz, points, W0, g0, b0, W1, g1, b1, W2, g2, b2):
    idx = _fps(jax.lax.stop_gradient(xyz), _NPOINT)
    new_xyz = _gather_pts(xyz, idx)
    gidx = _ball_query(_RADIUS, _NSAMPLE, jax.lax.stop_gradient(xyz),
                       jax.lax.stop_gradient(new_xyz))
    grouped_xyz = _gather_pts(xyz, gidx) - new_xyz[:, :, None, :]
    grouped_pts = _gather_pts(jnp.transpose(points, (0, 2, 1)), gidx)
    feats = jnp.concatenate([grouped_xyz, grouped_pts], axis=-1)
    x0 = feats.reshape(_M, _IN_CH + 3)
    x0 = jnp.pad(x0, ((0, 0), (0, _C - (_IN_CH + 3))))

    w0 = _pad_w(W0)
    w1 = _pad_w(W1)
    w2 = _pad_w(W2)

    row_spec = pl.BlockSpec((_BLK, _C), lambda i: (i, 0))
    full_spec = pl.BlockSpec((8, _C), lambda i: (0, 0))
    w_spec = pl.BlockSpec((_C, _C), lambda i: (0, 0))

    y0, st0 = pl.pallas_call(
        _mm_stats_kernel,
        grid=(_GRID,),
        in_specs=[row_spec, w_spec],
        out_specs=[row_spec, full_spec],
        out_shape=[jax.ShapeDtypeStruct((_M, _C), jnp.float32),
                   jax.ShapeDtypeStruct((8, _C), jnp.float32)],
    )(x0, w0)
    aff0 = _affine_from_stats(st0, _pad_vec(g0), _pad_vec(b0))

    y1, st1 = pl.pallas_call(
        _affine_mm_stats_kernel,
        grid=(_GRID,),
        in_specs=[row_spec, full_spec, w_spec],
        out_specs=[row_spec, full_spec],
        out_shape=[jax.ShapeDtypeStruct((_M, _C), jnp.float32),
                   jax.ShapeDtypeStruct((8, _C), jnp.float32)],
    )(y0, aff0, w1)
    aff1 = _affine_from_stats(st1, _pad_vec(g1), _pad_vec(b1))

    y2, st2 = pl.pallas_call(
        _affine_mm_stats_kernel,
        grid=(_GRID,),
        in_specs=[row_spec, full_spec, w_spec],
        out_specs=[row_spec, full_spec],
        out_shape=[jax.ShapeDtypeStruct((_M, _C), jnp.float32),
                   jax.ShapeDtypeStruct((8, _C), jnp.float32)],
    )(y1, aff1, w2)
    aff2 = _affine_from_stats(st2, _pad_vec(g2), _pad_vec(b2))

    pooled = pl.pallas_call(
        _affine_maxpool_kernel,
        grid=(_GRID,),
        in_specs=[row_spec, full_spec],
        out_specs=pl.BlockSpec((_BLK // _NSAMPLE, _C), lambda i: (i, 0)),
        out_shape=jax.ShapeDtypeStruct((_M // _NSAMPLE, _C), jnp.float32),
    )(y2, aff2)

    out = pooled.reshape(_B, _NPOINT, _C)
    return (new_xyz, jnp.transpose(out, (0, 2, 1)))


# speedup vs baseline: 1.2217x; 1.2217x over previous
import functools

import jax
import jax.numpy as jnp
import numpy as np
from jax.experimental import pallas as pl

_B = 4
_N = 8192
_NPOINT = 1024
_RADIUS = 0.2
_NSAMPLE = 32
_IN_CH = 64

_M = _B * _NPOINT * _NSAMPLE
_BLK = 1024
_C = 128
_GRID = _M // _BLK


def _sqdist(src, dst):
    d = -2.0 * jnp.matmul(src, jnp.swapaxes(dst, 1, 2))
    d = d + jnp.sum(src ** 2, -1)[:, :, None]
    d = d + jnp.sum(dst ** 2, -1)[:, None, :]
    return d


def _gather_pts(points, idx):
    return jax.vmap(lambda p, i: p[i])(points, idx)


def _fps_kernel(x_ref, o_ref):
    x = x_ref[...]
    iota_n = jax.lax.broadcasted_iota(jnp.int32, (1, _N), 1)
    iota_p = jax.lax.broadcasted_iota(jnp.int32, (1, _NPOINT), 1)

    def body(i, carry):
        dist, far, cent = carry
        cent = jnp.where(iota_p == i, far, cent)
        mask = (iota_n == far).astype(jnp.float32)
        cen = jnp.sum(x * mask, axis=1, keepdims=True)
        d = jnp.sum((x - cen) ** 2, axis=0, keepdims=True)
        dist = jnp.minimum(dist, d)
        m = jnp.max(dist)
        far = jnp.min(jnp.where(dist == m, iota_n, _N)).astype(jnp.int32)
        return dist, far, cent

    carry = (jnp.full((1, _N), 1e10, jnp.float32),
             jnp.int32(0),
             jnp.zeros((1, _NPOINT), jnp.int32))
    _, _, cent = jax.lax.fori_loop(0, _NPOINT, body, carry)
    o_ref[...] = jnp.broadcast_to(cent, (8, _NPOINT))


def _fps(xyz, npoint):
    b, n, _ = xyz.shape
    xt = jnp.zeros((b, 8, n), jnp.float32).at[:, :3, :].set(
        jnp.transpose(xyz, (0, 2, 1))).reshape(b * 8, n)
    cent = pl.pallas_call(
        _fps_kernel,
        grid=(b,),
        in_specs=[pl.BlockSpec((8, n), lambda i: (i, 0))],
        out_specs=pl.BlockSpec((8, npoint), lambda i: (i, 0)),
        out_shape=jax.ShapeDtypeStruct((b * 8, npoint), jnp.int32),
    )(xt)
    return cent.reshape(b, 8, npoint)[:, 0, :]


def _ball_query(radius, nsample, xyz, new_xyz):
    b, n, _ = xyz.shape
    s = new_xyz.shape[1]
    sqr = _sqdist(new_xyz, xyz)
    gidx = jnp.broadcast_to(jnp.arange(n, dtype=jnp.int32), (b, s, n))
    gidx = jnp.where(sqr > radius ** 2, n, gidx)
    gidx = jnp.sort(gidx, axis=-1)[:, :, :nsample]
    first = gidx[:, :, :1]
    first = jnp.where(first == n, 0, first)
    gidx = jnp.where(gidx == n, jnp.broadcast_to(first, gidx.shape), gidx)
    return gidx


def _mm_stats_kernel(x_ref, w_ref, y_ref, stats_ref):
    y = jnp.dot(x_ref[...], w_ref[...], preferred_element_type=jnp.float32)
    y_ref[...] = y

    @pl.when(pl.program_id(0) == 0)
    def _():
        stats_ref[...] = jnp.zeros_like(stats_ref)

    s1 = jnp.sum(y, axis=0, keepdims=True)
    s2 = jnp.sum(y * y, axis=0, keepdims=True)
    pad = jnp.zeros((6, _C), jnp.float32)
    stats_ref[...] += jnp.concatenate([s1, s2, pad], axis=0)


def _affine_mm_stats_kernel(x_ref, aff_ref, w_ref, y_ref, stats_ref):
    aff = aff_ref[...]
    x = jnp.maximum(x_ref[...] * aff[0:1, :] + aff[1:2, :], 0.0)
    y = jnp.dot(x, w_ref[...], preferred_element_type=jnp.float32)
    y_ref[...] = y

    @pl.when(pl.program_id(0) == 0)
    def _():
        stats_ref[...] = jnp.zeros_like(stats_ref)

    s1 = jnp.sum(y, axis=0, keepdims=True)
    s2 = jnp.sum(y * y, axis=0, keepdims=True)
    pad = jnp.zeros((6, _C), jnp.float32)
    stats_ref[...] += jnp.concatenate([s1, s2, pad], axis=0)


def _affine_maxpool_kernel(x_ref, aff_ref, o_ref):
    aff = aff_ref[...]
    x = jnp.maximum(x_ref[...] * aff[0:1, :] + aff[1:2, :], 0.0)
    x = x.reshape(_BLK // _NSAMPLE, _NSAMPLE, _C)
    o_ref[...] = jnp.max(x, axis=1)


def _affine_from_stats(stats, g, b):
    mean = stats[0] / _M
    var = stats[1] / _M - mean * mean
    scale = g / jnp.sqrt(var + 1e-5)
    shift = b - mean * scale
    return jnp.concatenate([scale[None, :], shift[None, :],
                            jnp.zeros((6, _C), jnp.float32)], axis=0)


def _pad_w(W):
    o, i = W.shape
    Wt = jnp.zeros((_C, _C), jnp.float32).at[:i, :o].set(W.T)
    return Wt


def _pad_vec(v):
    return jnp.zeros((_C,), jnp.float32).at[: v.shape[0]].set(v)


@functools.partial(jax.jit)
def kernel(xyz, points, W0, g0, b0, W1, g1, b1, W2, g2, b2):
    idx = _fps(jax.lax.stop_gradient(xyz), _NPOINT)
    new_xyz = _gather_pts(xyz, idx)
    gidx = _ball_query(_RADIUS, _NSAMPLE, jax.lax.stop_gradient(xyz),
                       jax.lax.stop_gradient(new_xyz))
    grouped_xyz = _gather_pts(xyz, gidx) - new_xyz[:, :, None, :]
    grouped_pts = _gather_pts(jnp.transpose(points, (0, 2, 1)), gidx)
    feats = jnp.concatenate([grouped_xyz, grouped_pts], axis=-1)
    x0 = feats.reshape(_M, _IN_CH + 3)
    x0 = jnp.pad(x0, ((0, 0), (0, _C - (_IN_CH + 3))))

    w0 = _pad_w(W0)
    w1 = _pad_w(W1)
    w2 = _pad_w(W2)

    row_spec = pl.BlockSpec((_BLK, _C), lambda i: (i, 0))
    full_spec = pl.BlockSpec((8, _C), lambda i: (0, 0))
    w_spec = pl.BlockSpec((_C, _C), lambda i: (0, 0))

    y0, st0 = pl.pallas_call(
        _mm_stats_kernel,
        grid=(_GRID,),
        in_specs=[row_spec, w_spec],
        out_specs=[row_spec, full_spec],
        out_shape=[jax.ShapeDtypeStruct((_M, _C), jnp.float32),
                   jax.ShapeDtypeStruct((8, _C), jnp.float32)],
    )(x0, w0)
    aff0 = _affine_from_stats(st0, _pad_vec(g0), _pad_vec(b0))

    y1, st1 = pl.pallas_call(
        _affine_mm_stats_kernel,
        grid=(_GRID,),
        in_specs=[row_spec, full_spec, w_spec],
        out_specs=[row_spec, full_spec],
        out_shape=[jax.ShapeDtypeStruct((_M, _C), jnp.float32),
                   jax.ShapeDtypeStruct((8, _C), jnp.float32)],
    )(y0, aff0, w1)
    aff1 = _affine_from_stats(st1, _pad_vec(g1), _pad_vec(b1))

    y2, st2 = pl.pallas_call(
        _affine_mm_stats_kernel,
        grid=(_GRID,),
        in_specs=[row_spec, full_spec, w_spec],
        out_specs=[row_spec, full_spec],
        out_shape=[jax.ShapeDtypeStruct((_M, _C), jnp.float32),
                   jax.ShapeDtypeStruct((8, _C), jnp.float32)],
    )(y1, aff1, w2)
    aff2 = _affine_from_stats(st2, _pad_vec(g2), _pad_vec(b2))

    pooled = pl.pallas_call(
        _affine_maxpool_kernel,
        grid=(_GRID,),
        in_specs=[row_spec, full_spec],
        out_specs=pl.BlockSpec((_BLK // _NSAMPLE, _C), lambda i: (i, 0)),
        out_shape=jax.ShapeDtypeStruct((_M // _NSAMPLE, _C), jnp.float32),
    )(y2, aff2)

    out = pooled.reshape(_B, _NPOINT, _C)
    return (new_xyz, jnp.transpose(out, (0, 2, 1)))
